# baseline (speedup 1.0000x reference)
import functools
from functools import partial

import jax
import jax.numpy as jnp
from jax import lax
from jax.experimental import pallas as pl
from jax.experimental.pallas import tpu as pltpu
from jax.experimental.pallas import tpu_sc as plsc

N = 10000
E = 320000
D = 128
H = 128
OUT = 16
G = 64
L = 3
S = 2

NC = 2
NS = 16
NW = NC * NS
CHUNK = 64
NCHUNK = 160
EPW = NCHUNK * CHUNK
EPPW = E // NW
PADW = EPW - EPPW
NPAD = 10112
ROWS_PER_TILE = NPAD // NS
TRASH_ROW = N

B_BLK = 1000
NBLK = N // B_BLK


NBUF = 4
IB = 16
NIB = NCHUNK // IB


SBC = 2 * IB
NSB = NCHUNK // SBC


def _seg_sum_body(h_hbm, col3_hbm, row3_hbm, zeros_hbm, out_hbm,
                  colb0, rowb0, colb1, rowb1,
                  row1_0, row1_1, row1_2, row1_3,
                  rows0, rows1, rows2, rows3,
                  sg0, sg1, sg2, sg3, ss0, ss1, ss2, ss3,
                  semi0, semi1, acc_sh):
    semg = (sg0, sg1, sg2, sg3)
    sems = (ss0, ss1, ss2, ss3)
    rows = (rows0, rows1, rows2, rows3)
    row1 = (row1_0, row1_1, row1_2, row1_3)
    colb = (colb0, colb1)
    rowb = (rowb0, rowb1)
    semi = (semi0, semi1)
    c = lax.axis_index("c")
    s = lax.axis_index("s")
    wid = s * NC + c
    me = col3_hbm.at[wid]
    mer = row3_hbm.at[wid]

    def fire_g(b, p, tb):
        pltpu.async_copy(h_hbm.at[colb[p].at[tb]], rows[b], semg[b])

    def wait_g(b, p, tb):
        pltpu.make_async_copy(h_hbm.at[colb[p].at[tb]], rows[b],
                              semg[b]).wait()

    def fire_s(b, p, tb):
        for k in range(CHUNK // 16):
            row1[b][pl.ds(k * 16, 16)] = rowb[p][tb, pl.ds(k * 16, 16)]
        pltpu.async_copy(rows[b], acc_sh.at[row1[b]], sems[b], add=True)

    def wait_s(b):
        pltpu.make_async_copy(rows[b], acc_sh.at[row1[b]], sems[b]).wait()

    def wait_i(p):
        pltpu.make_async_copy(me.at[pl.ds(0, IB)], colb[p], semi[p]).wait()
        pltpu.make_async_copy(mer.at[pl.ds(0, IB)], rowb[p], semi[p]).wait()

    def fire_i(p, off):
        pltpu.async_copy(me.at[pl.ds(off, IB)], colb[p], semi[p])
        pltpu.async_copy(mer.at[pl.ds(off, IB)], rowb[p], semi[p])

    def step(jj, t, first):
        b = t % NBUF
        p = (t // IB) % 2
        tb = t % IB
        wait_g(b, p, tb)
        fire_s(b, p, tb)
        b3 = (t + 3) % NBUF
        if not first:
            wait_s(b3)
        t3 = t + 3
        p3 = (t3 // IB) % 2
        tb3 = t3 % IB
        if t3 < SBC:
            if t == IB - 3:
                wait_i(p3)
            fire_g(b3, p3, tb3)
        else:
            @pl.when(jj + 3 < NCHUNK)
            def _():
                if t == SBC - 3:
                    wait_i(0)
                fire_g(b3, 0, tb3)
        if tb == IB - 1:
            go = pl.multiple_of(jj + 1 + IB, IB)

            @pl.when(go + IB <= NCHUNK)
            def _():
                fire_i(p, go)

    fire_i(0, 0)
    fire_i(1, IB)
    pltpu.sync_copy(zeros_hbm,
                    acc_sh.at[pl.ds(s * ROWS_PER_TILE, ROWS_PER_TILE)])
    plsc.subcore_barrier()

    wait_i(0)
    for b in range(NBUF - 1):
        fire_g(b, 0, b)
    for t in range(SBC):
        step(t, t, first=(t == 0))

    def _sb(sb, carry):
        j0 = sb * SBC
        for t in range(SBC):
            step(j0 + t, t, first=False)
        return carry

    lax.fori_loop(1, NSB, _sb, 0)
    wait_s((NCHUNK - 1) % NBUF)

    plsc.subcore_barrier()
    pltpu.sync_copy(acc_sh.at[pl.ds(s * ROWS_PER_TILE, ROWS_PER_TILE)],
                    out_hbm.at[c].at[pl.ds(s * ROWS_PER_TILE, ROWS_PER_TILE)])


@functools.cache
def _build_seg_sum():
    mesh = plsc.VectorSubcoreMesh(core_axis_name="c", subcore_axis_name="s",
                                  num_cores=NC, num_subcores=NS)
    return pl.kernel(
        _seg_sum_body,
        out_type=jax.ShapeDtypeStruct((NC, NPAD, H), jnp.float32),
        mesh=mesh,
        scratch_types=(
            [pltpu.VMEM((IB, CHUNK), jnp.int32) for _ in range(4)]
            + [pltpu.VMEM((CHUNK,), jnp.int32) for _ in range(NBUF)]
            + [pltpu.VMEM((CHUNK, H), jnp.float32) for _ in range(NBUF)]
            + [pltpu.SemaphoreType.DMA] * (2 * NBUF + 2)
            + [pltpu.VMEM_SHARED((NPAD, H), jnp.float32)]
        ),
    )


def _mlp0_body(x_ref, w_ref, b_ref, o_ref):
    t = jnp.dot(x_ref[...], w_ref[...], preferred_element_type=jnp.float32)
    o_ref[...] = jnp.maximum(t + b_ref[...], 0.0)


def _gin_mlp_body(p_ref, h_ref, eps_ref, w1_ref, b1_ref, w2_ref, b2_ref,
                  o_ref):
    t = p_ref[0] + p_ref[1] + (1.0 + eps_ref[0, 0]) * h_ref[...]
    t = jnp.maximum(
        jnp.dot(t, w1_ref[...], preferred_element_type=jnp.float32)
        + b1_ref[...], 0.0)
    o_ref[...] = jnp.maximum(
        jnp.dot(t, w2_ref[...], preferred_element_type=jnp.float32)
        + b2_ref[...], 0.0)


def _pool_head_body(p_ref, h_ref, eps_ref, wm1_ref, bm1_ref, wm2_ref,
                    bm2_ref, seg_ref, w1_ref, b1_ref, w2_ref, b2_ref,
                    o_ref, acc_ref):
    i = pl.program_id(0)

    @pl.when(i == 0)
    def _():
        acc_ref[...] = jnp.zeros_like(acc_ref)

    t = p_ref[0] + p_ref[1] + (1.0 + eps_ref[0, 0]) * h_ref[...]
    t = jnp.maximum(
        jnp.dot(t, wm1_ref[...], preferred_element_type=jnp.float32)
        + bm1_ref[...], 0.0)
    t = jnp.maximum(
        jnp.dot(t, wm2_ref[...], preferred_element_type=jnp.float32)
        + bm2_ref[...], 0.0)

    seg = seg_ref[0, 0, :]
    onehot = (seg[None, :]
              == lax.broadcasted_iota(jnp.int32, (G, B_BLK), 0)
              ).astype(jnp.float32)
    acc_ref[...] += jnp.dot(onehot, t, preferred_element_type=jnp.float32)

    @pl.when(i == pl.num_programs(0) - 1)
    def _():
        g = jnp.maximum(
            jnp.dot(acc_ref[...], w1_ref[...],
                    preferred_element_type=jnp.float32) + b1_ref[...], 0.0)
        o = jnp.dot(g, w2_ref[...],
                    preferred_element_type=jnp.float32) + b2_ref[...]
        m = jnp.max(o, axis=-1, keepdims=True)
        e = jnp.exp(o - m)
        o_ref[...] = e / jnp.sum(e, axis=-1, keepdims=True)


def _full(shape):
    return pl.BlockSpec(shape, lambda i: tuple(0 for _ in shape))


_mlp0 = pl.pallas_call(
    _mlp0_body,
    grid=(NBLK,),
    in_specs=[
        pl.BlockSpec((B_BLK, D), lambda i: (i, 0)),
        _full((D, H)),
        _full((1, H)),
    ],
    out_specs=pl.BlockSpec((B_BLK, H), lambda i: (i, 0)),
    out_shape=jax.ShapeDtypeStruct((N, H), jnp.float32),
)

_gin_mlp = pl.pallas_call(
    _gin_mlp_body,
    grid=(NBLK,),
    in_specs=[
        pl.BlockSpec((NC, B_BLK, H), lambda i: (0, i, 0)),
        pl.BlockSpec((B_BLK, H), lambda i: (i, 0)),
        _full((1, 1)),
        _full((H, H)),
        _full((1, H)),
        _full((H, H)),
        _full((1, H)),
    ],
    out_specs=pl.BlockSpec((B_BLK, H), lambda i: (i, 0)),
    out_shape=jax.ShapeDtypeStruct((N, H), jnp.float32),
)

_pool_head = pl.pallas_call(
    _pool_head_body,
    grid=(NBLK,),
    in_specs=[
        pl.BlockSpec((NC, B_BLK, H), lambda i: (0, i, 0)),
        pl.BlockSpec((B_BLK, H), lambda i: (i, 0)),
        _full((1, 1)),
        _full((H, H)),
        _full((1, H)),
        _full((H, H)),
        _full((1, H)),
        pl.BlockSpec((1, 1, B_BLK), lambda i: (i, 0, 0)),
        _full((H, H)),
        _full((1, H)),
        _full((H, OUT)),
        _full((1, OUT)),
    ],
    out_specs=_full((G, OUT)),
    out_shape=jax.ShapeDtypeStruct((G, OUT), jnp.float32),
    scratch_shapes=[pltpu.VMEM((G, H), jnp.float32)],
)


def kernel(x, edge_index, batch, eps, W_first, b_first, W_mlp, b_mlp,
           W_lin1, b_lin1, W_lin2, b_lin2):
    pad_rows = TRASH_ROW + (jnp.arange(PADW, dtype=jnp.int32) % (NPAD - N))
    pad_cols = (jnp.arange(PADW, dtype=jnp.int32) * 37) % N
    row = jnp.concatenate(
        [edge_index[0].reshape(NW, EPPW),
         jnp.broadcast_to(pad_rows, (NW, PADW))], axis=1
    ).reshape(NW, NCHUNK, CHUNK)
    col = jnp.concatenate(
        [edge_index[1].reshape(NW, EPPW),
         jnp.broadcast_to(pad_cols, (NW, PADW))], axis=1
    ).reshape(NW, NCHUNK, CHUNK)
    zeros = jnp.zeros((ROWS_PER_TILE, H), jnp.float32)

    seg_sum = _build_seg_sum()
    h = _mlp0(x, W_first, b_first.reshape(1, H))
    for l in range(L - 1):
        parts = seg_sum(h, col, row, zeros)
        h = _gin_mlp(parts, h, eps[l].reshape(1, 1),
                     W_mlp[l, 0], b_mlp[l, 0].reshape(1, H),
                     W_mlp[l, 1], b_mlp[l, 1].reshape(1, H))
    parts = seg_sum(h, col, row, zeros)
    return _pool_head(parts, h, eps[L - 1].reshape(1, 1),
                      W_mlp[L - 1, 0], b_mlp[L - 1, 0].reshape(1, H),
                      W_mlp[L - 1, 1], b_mlp[L - 1, 1].reshape(1, H),
                      batch.reshape(NBLK, 1, B_BLK),
                      W_lin1, b_lin1.reshape(1, H),
                      W_lin2, b_lin2.reshape(1, OUT))


# speedup vs baseline: 1.0220x; 1.0220x over previous
import functools
from functools import partial

import jax
import jax.numpy as jnp
from jax import lax
from jax.experimental import pallas as pl
from jax.experimental.pallas import tpu as pltpu
from jax.experimental.pallas import tpu_sc as plsc

N = 10000
E = 320000
D = 128
H = 128
OUT = 16
G = 64
L = 3
S = 2

NC = 2
NS = 16
NW = NC * NS
CHUNK = 80
NCHUNK = 128
EPW = NCHUNK * CHUNK
EPPW = E // NW
PADW = EPW - EPPW
NPAD = 10112
ROWS_PER_TILE = NPAD // NS
TRASH_ROW = N

B_BLK = 1000
NBLK = N // B_BLK


NBUF = 4
IB = 8
NIB = NCHUNK // IB


SBC = 2 * IB
NSB = NCHUNK // SBC


def _seg_sum_body(h_hbm, col3_hbm, row3_hbm, zeros_hbm, out_hbm,
                  colb0, colb1,
                  row1_0, row1_1, row1_2, row1_3,
                  rows0, rows1, rows2, rows3,
                  sg0, sg1, sg2, sg3, ss0, ss1, ss2, ss3,
                  sr0, sr1, sr2, sr3,
                  semi0, semi1, acc_sh):
    semg = (sg0, sg1, sg2, sg3)
    sems = (ss0, ss1, ss2, ss3)
    semr = (sr0, sr1, sr2, sr3)
    rows = (rows0, rows1, rows2, rows3)
    row1 = (row1_0, row1_1, row1_2, row1_3)
    colb = (colb0, colb1)
    semi = (semi0, semi1)
    c = lax.axis_index("c")
    s = lax.axis_index("s")
    wid = s * NC + c
    me = col3_hbm.at[wid]
    mer = row3_hbm.at[wid]

    def fire_g(b, p, tb, jj):
        pltpu.async_copy(h_hbm.at[colb[p].at[tb]], rows[b], semg[b])
        pltpu.async_copy(mer.at[jj], row1[b], semr[b])

    def wait_g(b, p, tb):
        pltpu.make_async_copy(h_hbm.at[colb[p].at[tb]], rows[b],
                              semg[b]).wait()

    def fire_s(b):
        pltpu.make_async_copy(mer.at[0], row1[b], semr[b]).wait()
        pltpu.async_copy(rows[b], acc_sh.at[row1[b]], sems[b], add=True)

    def wait_s(b):
        pltpu.make_async_copy(rows[b], acc_sh.at[row1[b]], sems[b]).wait()

    def wait_i(p):
        pltpu.make_async_copy(me.at[pl.ds(0, IB)], colb[p], semi[p]).wait()

    def fire_i(p, off):
        pltpu.async_copy(me.at[pl.ds(off, IB)], colb[p], semi[p])

    def step(jj, t, first):
        b = t % NBUF
        p = (t // IB) % 2
        tb = t % IB
        wait_g(b, p, tb)
        fire_s(b)
        b3 = (t + 3) % NBUF
        if not first:
            wait_s(b3)
        t3 = t + 3
        p3 = (t3 // IB) % 2
        tb3 = t3 % IB
        if t3 < SBC:
            if t == IB - 3:
                wait_i(p3)
            fire_g(b3, p3, tb3, jj + 3)
        else:
            @pl.when(jj + 3 < NCHUNK)
            def _():
                if t == SBC - 3:
                    wait_i(0)
                fire_g(b3, 0, tb3, jj + 3)
        if tb == IB - 1:
            go = pl.multiple_of(jj + 1 + IB, IB)

            @pl.when(go + IB <= NCHUNK)
            def _():
                fire_i(p, go)

    fire_i(0, 0)
    fire_i(1, IB)
    pltpu.sync_copy(zeros_hbm,
                    acc_sh.at[pl.ds(s * ROWS_PER_TILE, ROWS_PER_TILE)])
    plsc.subcore_barrier()

    wait_i(0)
    for b in range(NBUF - 1):
        fire_g(b, 0, b, b)
    for t in range(SBC):
        step(t, t, first=(t == 0))

    def _sb(sb, carry):
        j0 = sb * SBC
        for t in range(SBC):
            step(j0 + t, t, first=False)
        return carry

    lax.fori_loop(1, NSB, _sb, 0)
    wait_s((NCHUNK - 1) % NBUF)

    plsc.subcore_barrier()
    pltpu.sync_copy(acc_sh.at[pl.ds(s * ROWS_PER_TILE, ROWS_PER_TILE)],
                    out_hbm.at[c].at[pl.ds(s * ROWS_PER_TILE, ROWS_PER_TILE)])


@functools.cache
def _build_seg_sum():
    mesh = plsc.VectorSubcoreMesh(core_axis_name="c", subcore_axis_name="s",
                                  num_cores=NC, num_subcores=NS)
    return pl.kernel(
        _seg_sum_body,
        out_type=jax.ShapeDtypeStruct((NC, NPAD, H), jnp.float32),
        mesh=mesh,
        scratch_types=(
            [pltpu.VMEM((IB, CHUNK), jnp.int32) for _ in range(2)]
            + [pltpu.VMEM((CHUNK,), jnp.int32) for _ in range(NBUF)]
            + [pltpu.VMEM((CHUNK, H), jnp.float32) for _ in range(NBUF)]
            + [pltpu.SemaphoreType.DMA] * (3 * NBUF + 2)
            + [pltpu.VMEM_SHARED((NPAD, H), jnp.float32)]
        ),
    )


def _mlp0_body(x_ref, w_ref, b_ref, o_ref):
    t = jnp.dot(x_ref[...], w_ref[...], preferred_element_type=jnp.float32)
    o_ref[...] = jnp.maximum(t + b_ref[...], 0.0)


def _gin_mlp_body(p_ref, h_ref, eps_ref, w1_ref, b1_ref, w2_ref, b2_ref,
                  o_ref):
    t = p_ref[0] + p_ref[1] + (1.0 + eps_ref[0, 0]) * h_ref[...]
    t = jnp.maximum(
        jnp.dot(t, w1_ref[...], preferred_element_type=jnp.float32)
        + b1_ref[...], 0.0)
    o_ref[...] = jnp.maximum(
        jnp.dot(t, w2_ref[...], preferred_element_type=jnp.float32)
        + b2_ref[...], 0.0)


def _pool_head_body(p_ref, h_ref, eps_ref, wm1_ref, bm1_ref, wm2_ref,
                    bm2_ref, seg_ref, w1_ref, b1_ref, w2_ref, b2_ref,
                    o_ref, acc_ref):
    i = pl.program_id(0)

    @pl.when(i == 0)
    def _():
        acc_ref[...] = jnp.zeros_like(acc_ref)

    t = p_ref[0] + p_ref[1] + (1.0 + eps_ref[0, 0]) * h_ref[...]
    t = jnp.maximum(
        jnp.dot(t, wm1_ref[...], preferred_element_type=jnp.float32)
        + bm1_ref[...], 0.0)
    t = jnp.maximum(
        jnp.dot(t, wm2_ref[...], preferred_element_type=jnp.float32)
        + bm2_ref[...], 0.0)

    seg = seg_ref[0, 0, :]
    onehot = (seg[None, :]
              == lax.broadcasted_iota(jnp.int32, (G, B_BLK), 0)
              ).astype(jnp.float32)
    acc_ref[...] += jnp.dot(onehot, t, preferred_element_type=jnp.float32)

    @pl.when(i == pl.num_programs(0) - 1)
    def _():
        g = jnp.maximum(
            jnp.dot(acc_ref[...], w1_ref[...],
                    preferred_element_type=jnp.float32) + b1_ref[...], 0.0)
        o = jnp.dot(g, w2_ref[...],
                    preferred_element_type=jnp.float32) + b2_ref[...]
        m = jnp.max(o, axis=-1, keepdims=True)
        e = jnp.exp(o - m)
        o_ref[...] = e / jnp.sum(e, axis=-1, keepdims=True)


def _full(shape):
    return pl.BlockSpec(shape, lambda i: tuple(0 for _ in shape))


_mlp0 = pl.pallas_call(
    _mlp0_body,
    grid=(NBLK,),
    in_specs=[
        pl.BlockSpec((B_BLK, D), lambda i: (i, 0)),
        _full((D, H)),
        _full((1, H)),
    ],
    out_specs=pl.BlockSpec((B_BLK, H), lambda i: (i, 0)),
    out_shape=jax.ShapeDtypeStruct((N, H), jnp.float32),
)

_gin_mlp = pl.pallas_call(
    _gin_mlp_body,
    grid=(NBLK,),
    in_specs=[
        pl.BlockSpec((NC, B_BLK, H), lambda i: (0, i, 0)),
        pl.BlockSpec((B_BLK, H), lambda i: (i, 0)),
        _full((1, 1)),
        _full((H, H)),
        _full((1, H)),
        _full((H, H)),
        _full((1, H)),
    ],
    out_specs=pl.BlockSpec((B_BLK, H), lambda i: (i, 0)),
    out_shape=jax.ShapeDtypeStruct((N, H), jnp.float32),
)

_pool_head = pl.pallas_call(
    _pool_head_body,
    grid=(NBLK,),
    in_specs=[
        pl.BlockSpec((NC, B_BLK, H), lambda i: (0, i, 0)),
        pl.BlockSpec((B_BLK, H), lambda i: (i, 0)),
        _full((1, 1)),
        _full((H, H)),
        _full((1, H)),
        _full((H, H)),
        _full((1, H)),
        pl.BlockSpec((1, 1, B_BLK), lambda i: (i, 0, 0)),
        _full((H, H)),
        _full((1, H)),
        _full((H, OUT)),
        _full((1, OUT)),
    ],
    out_specs=_full((G, OUT)),
    out_shape=jax.ShapeDtypeStruct((G, OUT), jnp.float32),
    scratch_shapes=[pltpu.VMEM((G, H), jnp.float32)],
)


def kernel(x, edge_index, batch, eps, W_first, b_first, W_mlp, b_mlp,
           W_lin1, b_lin1, W_lin2, b_lin2):
    pad_rows = TRASH_ROW + (jnp.arange(PADW, dtype=jnp.int32) % (NPAD - N))
    pad_cols = (jnp.arange(PADW, dtype=jnp.int32) * 37) % N
    row = jnp.concatenate(
        [edge_index[0].reshape(NW, EPPW),
         jnp.broadcast_to(pad_rows, (NW, PADW))], axis=1
    ).reshape(NW, NCHUNK, CHUNK)
    col = jnp.concatenate(
        [edge_index[1].reshape(NW, EPPW),
         jnp.broadcast_to(pad_cols, (NW, PADW))], axis=1
    ).reshape(NW, NCHUNK, CHUNK)
    zeros = jnp.zeros((ROWS_PER_TILE, H), jnp.float32)

    seg_sum = _build_seg_sum()
    h = _mlp0(x, W_first, b_first.reshape(1, H))
    for l in range(L - 1):
        parts = seg_sum(h, col, row, zeros)
        h = _gin_mlp(parts, h, eps[l].reshape(1, 1),
                     W_mlp[l, 0], b_mlp[l, 0].reshape(1, H),
                     W_mlp[l, 1], b_mlp[l, 1].reshape(1, H))
    parts = seg_sum(h, col, row, zeros)
    return _pool_head(parts, h, eps[L - 1].reshape(1, 1),
                      W_mlp[L - 1, 0], b_mlp[L - 1, 0].reshape(1, H),
                      W_mlp[L - 1, 1], b_mlp[L - 1, 1].reshape(1, H),
                      batch.reshape(NBLK, 1, B_BLK),
                      W_lin1, b_lin1.reshape(1, H),
                      W_lin2, b_lin2.reshape(1, OUT))
